# Initial kernel scaffold; baseline (speedup 1.0000x reference)
import jax
import jax.numpy as jnp
from jax.experimental import pallas as pl


def kernel(x, pe_table):
    raise NotImplementedError("write your pallas kernel here")


# speedup vs baseline: 2.2996x; 2.2996x over previous
import jax
import jax.numpy as jnp
from jax.experimental import pallas as pl


def kernel(x, pe_table):
    B, S, D = x.shape
    BS = 512

    def body(pe_ref, o_ref):
        o_ref[...] = jnp.broadcast_to(pe_ref[...][None], o_ref.shape)

    out = pl.pallas_call(
        body,
        grid=(S // BS,),
        in_specs=[pl.BlockSpec((BS, D), lambda i: (i, 0))],
        out_specs=pl.BlockSpec((B, BS, D), lambda i: (0, i, 0)),
        out_shape=jax.ShapeDtypeStruct((B, S, D), pe_table.dtype),
    )(pe_table[:S])
    return out


# speedup vs baseline: 2.3686x; 1.0300x over previous
import jax
import jax.numpy as jnp
from jax.experimental import pallas as pl


def kernel(x, pe_table):
    B, S, D = x.shape
    BS = 1024

    def body(pe_ref, o_ref):
        o_ref[...] = jnp.broadcast_to(pe_ref[...][None], o_ref.shape)

    out = pl.pallas_call(
        body,
        grid=(S // BS,),
        in_specs=[pl.BlockSpec((BS, D), lambda i: (i, 0))],
        out_specs=pl.BlockSpec((B, BS, D), lambda i: (0, i, 0)),
        out_shape=jax.ShapeDtypeStruct((B, S, D), pe_table.dtype),
    )(pe_table[:S])
    return out


# speedup vs baseline: 2.4288x; 1.0254x over previous
import jax
import jax.numpy as jnp
from jax.experimental import pallas as pl
from jax.experimental.pallas import tpu as pltpu


def kernel(x, pe_table):
    B, S, D = x.shape
    NCHUNK = 8
    CS = S // NCHUNK

    def body(pe_hbm, o_hbm, vmem, in_sems, out_sems):
        def in_copy(c):
            return pltpu.make_async_copy(
                pe_hbm.at[pl.ds(c * CS, CS)],
                vmem.at[pl.ds(c * CS, CS)],
                in_sems.at[c],
            )

        def out_copy(c, b):
            return pltpu.make_async_copy(
                vmem.at[pl.ds(c * CS, CS)],
                o_hbm.at[b, pl.ds(c * CS, CS)],
                out_sems.at[c, b],
            )

        for c in range(NCHUNK):
            in_copy(c).start()
        for c in range(NCHUNK):
            in_copy(c).wait()
            for b in range(B):
                out_copy(c, b).start()
        for c in range(NCHUNK):
            for b in range(B):
                out_copy(c, b).wait()

    out = pl.pallas_call(
        body,
        in_specs=[pl.BlockSpec(memory_space=pl.ANY)],
        out_specs=pl.BlockSpec(memory_space=pl.ANY),
        out_shape=jax.ShapeDtypeStruct((B, S, D), pe_table.dtype),
        scratch_shapes=[
            pltpu.VMEM((S, D), pe_table.dtype),
            pltpu.SemaphoreType.DMA((NCHUNK,)),
            pltpu.SemaphoreType.DMA((NCHUNK, B)),
        ],
    )(pe_table[:S])
    return out
